# Initial kernel scaffold; baseline (speedup 1.0000x reference)
import jax
import jax.numpy as jnp
from jax.experimental import pallas as pl


def kernel(x, edge_index, W1, b1, W2, b2):
    raise NotImplementedError("write your pallas kernel here")


# speedup vs baseline: 11.0041x; 11.0041x over previous
import functools

import jax
import jax.numpy as jnp
from jax import lax
from jax.experimental import pallas as pl
from jax.experimental.pallas import tpu as pltpu
from jax.experimental.pallas import tpu_sc as plsc

NC = 2
NS = 16
CH = 128


def _fill1d(ref, n, val):
    def body(i, _):
        ref[pl.ds(i * 16, 16)] = jnp.full((16,), val, jnp.float32)
        return 0
    lax.fori_loop(0, n // 16, body, 0)


def _fill2d(ref, rows, val):
    def body(i, _):
        for k in range(8):
            ref[i, pl.ds(k * 16, 16)] = jnp.full((16,), val, jnp.float32)
        return 0
    lax.fori_loop(0, rows, body, 0)


def _sc_degree(eidx, npad, nper):
    js, sup = eidx.shape[2], eidx.shape[3]
    mesh = plsc.VectorSubcoreMesh(core_axis_name="c", subcore_axis_name="s")

    @functools.partial(
        pl.kernel, mesh=mesh,
        out_type=jax.ShapeDtypeStruct((NC, npad), jnp.float32),
        scratch_types=[
            pltpu.VMEM((sup, 2, CH), jnp.int32),
            pltpu.VMEM((CH,), jnp.float32),
            pltpu.VMEM((nper,), jnp.float32),
            pltpu.VMEM_SHARED((npad,), jnp.float32),
            pltpu.SemaphoreType.DMA,
        ])
    def k(eidx_hbm, out_hbm, idxs, onesv, zv, acc, sem):
        c = lax.axis_index("c")
        s = lax.axis_index("s")
        _fill1d(onesv, CH, 1.0)
        _fill1d(zv, nper, 0.0)
        pltpu.sync_copy(zv, acc.at[pl.ds(s * nper, nper)])
        plsc.subcore_barrier()

        def body(t, _):
            pltpu.sync_copy(eidx_hbm.at[c, s, t], idxs)
            for k in range(sup):
                pltpu.sync_copy(onesv, acc.at[idxs.at[k, 0]], add=True)
            return 0
        lax.fori_loop(0, js, body, 0)
        plsc.subcore_barrier()
        pltpu.sync_copy(acc.at[pl.ds(s * nper, nper)],
                        out_hbm.at[c, pl.ds(s * nper, nper)])

    return k(eidx)


def _sc_propagate(yp, eidx, dinvp, npad, nper, with_scalar):
    js, sup = eidx.shape[2], eidx.shape[3]
    d = yp.shape[1]
    mesh = plsc.VectorSubcoreMesh(core_axis_name="c", subcore_axis_name="s")
    out_type = [jax.ShapeDtypeStruct((NC, npad, d), jnp.float32)]
    if with_scalar:
        out_type.append(jax.ShapeDtypeStruct((NC, npad), jnp.float32))
    scratch = [
        pltpu.VMEM((sup, 2, CH), jnp.int32),
        pltpu.VMEM((CH, d), jnp.float32),
        pltpu.VMEM((CH, d), jnp.float32),
        pltpu.VMEM_SHARED((npad, d), jnp.float32),
        pltpu.SemaphoreType.DMA,
        pltpu.SemaphoreType.DMA,
        pltpu.SemaphoreType.DMA,
    ]
    if with_scalar:
        scratch += [
            pltpu.VMEM((CH,), jnp.float32),
            pltpu.VMEM((nper,), jnp.float32),
            pltpu.VMEM_SHARED((npad,), jnp.float32),
        ]

    def k(*refs):
        if with_scalar:
            (y_hbm, eidx_hbm, dinv_hbm, p_hbm, sa_hbm,
             idxs, bufa, bufb, acc, sema, semb, semc,
             dchunk, zv, sacc) = refs
        else:
            (y_hbm, eidx_hbm, p_hbm,
             idxs, bufa, bufb, acc, sema, semb, semc) = refs
        c = lax.axis_index("c")
        s = lax.axis_index("s")
        _fill2d(bufa, CH, 0.0)
        for i in range(nper // CH):
            pltpu.sync_copy(bufa, acc.at[pl.ds(s * nper + i * CH, CH)])
        if with_scalar:
            _fill1d(zv, nper, 0.0)
            pltpu.sync_copy(zv, sacc.at[pl.ds(s * nper, nper)])
        plsc.subcore_barrier()

        def gather(k_, buf, sem):
            return pltpu.async_copy(y_hbm.at[idxs.at[k_, 0]], buf, sem)

        def scatter(k_, buf):
            pltpu.sync_copy(buf, acc.at[idxs.at[k_, 1]], add=True)
            if with_scalar:
                pltpu.async_copy(dinv_hbm.at[idxs.at[k_, 0]], dchunk,
                                 semc).wait()
                pltpu.sync_copy(dchunk, sacc.at[idxs.at[k_, 1]], add=True)

        def body(t, _):
            pltpu.sync_copy(eidx_hbm.at[c, s, t], idxs)
            gather(0, bufa, sema).wait()
            for k_ in range(0, sup, 2):
                gb = gather(k_ + 1, bufb, semb)
                scatter(k_, bufa)
                gb.wait()
                if k_ + 2 < sup:
                    ga = gather(k_ + 2, bufa, sema)
                    scatter(k_ + 1, bufb)
                    ga.wait()
                else:
                    scatter(k_ + 1, bufb)
            return 0
        lax.fori_loop(0, js, body, 0)
        plsc.subcore_barrier()
        for i in range(nper // CH):
            r0 = s * nper + i * CH
            pltpu.sync_copy(acc.at[pl.ds(r0, CH)], p_hbm.at[c, pl.ds(r0, CH)])
        if with_scalar:
            pltpu.sync_copy(sacc.at[pl.ds(s * nper, nper)],
                            sa_hbm.at[c, pl.ds(s * nper, nper)])

    kfn = functools.partial(pl.kernel, mesh=mesh, out_type=out_type,
                            scratch_types=scratch)(k)
    if with_scalar:
        return kfn(yp, eidx, dinvp)
    return kfn(yp, eidx)


def _tc_scale(dinvp, xp, npad, d, bn=1024):
    g = npad // bn

    def body(dv, xr, o):
        o[...] = dv[...][:, None] * xr[...]

    return pl.pallas_call(
        body, grid=(g,),
        in_specs=[pl.BlockSpec((bn,), lambda i: (i,)),
                  pl.BlockSpec((bn, d), lambda i: (i, 0))],
        out_specs=pl.BlockSpec((bn, d), lambda i: (i, 0)),
        out_shape=jax.ShapeDtypeStruct((npad, d), jnp.float32),
    )(dinvp, xp)


def _tc_mlp(p0, p1, xp, dinvp, dinv2p, sp, W1, b1, W2, b2, npad, bn=1024):
    g = npad // bn
    d_in = xp.shape[1]
    d_h, d_out = W1.shape[0], W2.shape[0]

    def body(p0r, p1r, xr, dvr, d2r, sr, w1r, b1r, w2r, b2r, g_ref, y1_ref):
        dv = dvr[...][:, None]
        d2 = d2r[...][:, None]
        t = dv * (p0r[...] + p1r[...]) + d2 * xr[...]
        h = lax.dot_general(t, w1r[...], (((1,), (1,)), ((), ())),
                            preferred_element_type=jnp.float32)
        h = jnp.maximum(h + sr[...][:, None] * b1r[...][None, :], 0.0)
        gm = lax.dot_general(h, w2r[...], (((1,), (1,)), ((), ())),
                             preferred_element_type=jnp.float32)
        gm = gm + b2r[...][None, :]
        g_ref[...] = gm
        y1_ref[...] = dv * gm

    return pl.pallas_call(
        body, grid=(g,),
        in_specs=[pl.BlockSpec((bn, d_in), lambda i: (i, 0)),
                  pl.BlockSpec((bn, d_in), lambda i: (i, 0)),
                  pl.BlockSpec((bn, d_in), lambda i: (i, 0)),
                  pl.BlockSpec((bn,), lambda i: (i,)),
                  pl.BlockSpec((bn,), lambda i: (i,)),
                  pl.BlockSpec((bn,), lambda i: (i,)),
                  pl.BlockSpec((d_h, d_in), lambda i: (0, 0)),
                  pl.BlockSpec((d_h,), lambda i: (0,)),
                  pl.BlockSpec((d_out, d_h), lambda i: (0, 0)),
                  pl.BlockSpec((d_out,), lambda i: (0,))],
        out_specs=[pl.BlockSpec((bn, d_out), lambda i: (i, 0)),
                   pl.BlockSpec((bn, d_out), lambda i: (i, 0))],
        out_shape=[jax.ShapeDtypeStruct((npad, d_out), jnp.float32),
                   jax.ShapeDtypeStruct((npad, d_out), jnp.float32)],
    )(p0, p1, xp, dinvp, dinv2p, sp, W1, b1, W2, b2)


def _tc_combine(p0, p1, gm, dinvp, dinv2p, npad, d, bn=1024):
    g = npad // bn

    def body(p0r, p1r, gr, dvr, d2r, o):
        o[...] = (dvr[...][:, None] * (p0r[...] + p1r[...])
                  + d2r[...][:, None] * gr[...])

    return pl.pallas_call(
        body, grid=(g,),
        in_specs=[pl.BlockSpec((bn, d), lambda i: (i, 0)),
                  pl.BlockSpec((bn, d), lambda i: (i, 0)),
                  pl.BlockSpec((bn, d), lambda i: (i, 0)),
                  pl.BlockSpec((bn,), lambda i: (i,)),
                  pl.BlockSpec((bn,), lambda i: (i,))],
        out_specs=pl.BlockSpec((bn, d), lambda i: (i, 0)),
        out_shape=jax.ShapeDtypeStruct((npad, d), jnp.float32),
    )(p0, p1, gm, dinvp, dinv2p)


def kernel(x, edge_index, W1, b1, W2, b2):
    n, d_in = x.shape
    e = edge_index.shape[1]
    nw = NC * NS
    sup = 8
    assert e % nw == 0
    ept = e // nw
    js = -(-ept // (CH * sup))
    pt = js * sup * CH
    npad = -(-n // (NS * CH)) * (NS * CH)
    nper = npad // NS

    row = edge_index[0].reshape(nw, ept)
    col = edge_index[1].reshape(nw, ept)
    pad = ((0, 0), (0, pt - ept))
    colp = jnp.pad(col, pad, constant_values=n).reshape(nw, js * sup, CH)
    rowp = jnp.pad(row, pad, constant_values=n).reshape(nw, js * sup, CH)
    eidx = jnp.stack([colp, rowp], axis=2)
    eidx = eidx.reshape(NC, NS, js, sup, 2, CH)

    xp = jnp.pad(x, ((0, npad - n), (0, 0)))

    cnt = _sc_degree(eidx, npad, nper)
    deg = cnt[0, :n] + cnt[1, :n] + 1.0
    dinv = lax.rsqrt(deg)
    dinv2 = 1.0 / deg
    dinvp = jnp.pad(dinv, (0, npad - n))
    dinv2p = jnp.pad(dinv2, (0, npad - n))

    y0 = _tc_scale(dinvp, xp, npad, d_in)
    p, sa = _sc_propagate(y0, eidx, dinvp, npad, nper, True)
    s = dinv * (sa[0, :n] + sa[1, :n]) + dinv2
    sp = jnp.pad(s, (0, npad - n))

    g, y1 = _tc_mlp(p[0], p[1], xp, dinvp, dinv2p, sp, W1, b1, W2, b2, npad)

    p2 = _sc_propagate(y1, eidx, None, npad, nper, False)[0]
    out = _tc_combine(p2[0], p2[1], g, dinvp, dinv2p, npad, x.shape[1])
    return out[:n]


# speedup vs baseline: 11.2161x; 1.0193x over previous
import functools

import jax
import jax.numpy as jnp
from jax import lax
from jax.experimental import pallas as pl
from jax.experimental.pallas import tpu as pltpu
from jax.experimental.pallas import tpu_sc as plsc

NC = 2
NS = 16
CH = 128


def _fill1d(ref, n, val):
    def body(i, _):
        ref[pl.ds(i * 16, 16)] = jnp.full((16,), val, jnp.float32)
        return 0
    lax.fori_loop(0, n // 16, body, 0)


def _fill2d(ref, rows, val):
    def body(i, _):
        for k in range(8):
            ref[i, pl.ds(k * 16, 16)] = jnp.full((16,), val, jnp.float32)
        return 0
    lax.fori_loop(0, rows, body, 0)


def _sc_degree(eidx, npad, nper):
    js, sup = eidx.shape[2], eidx.shape[3]
    hh = CH // 2
    mesh = plsc.VectorSubcoreMesh(core_axis_name="c", subcore_axis_name="s")

    @functools.partial(
        pl.kernel, mesh=mesh,
        out_type=jax.ShapeDtypeStruct((NC, npad), jnp.float32),
        scratch_types=[
            pltpu.VMEM((sup, 2, 2, hh), jnp.int32),
            pltpu.VMEM((hh,), jnp.float32),
            pltpu.VMEM((nper,), jnp.float32),
            pltpu.VMEM_SHARED((npad,), jnp.float32),
            pltpu.SemaphoreType.DMA,
        ])
    def k(eidx_hbm, out_hbm, idxs, onesv, zv, acc, sem):
        c = lax.axis_index("c")
        s = lax.axis_index("s")
        _fill1d(onesv, hh, 1.0)
        _fill1d(zv, nper, 0.0)
        pltpu.sync_copy(zv, acc.at[pl.ds(s * nper, nper)])
        plsc.subcore_barrier()

        def body(t, _):
            pltpu.sync_copy(eidx_hbm.at[c, s, t], idxs)
            for k in range(sup):
                for h in range(2):
                    pltpu.sync_copy(onesv, acc.at[idxs.at[k, 0, h]],
                                    add=True)
            return 0
        lax.fori_loop(0, js, body, 0)
        plsc.subcore_barrier()
        pltpu.sync_copy(acc.at[pl.ds(s * nper, nper)],
                        out_hbm.at[c, pl.ds(s * nper, nper)])

    return k(eidx)


def _sc_propagate(yp, eidx, dinvp, npad, nper, with_scalar):
    js, sup = eidx.shape[2], eidx.shape[3]
    d = yp.shape[1]
    mesh = plsc.VectorSubcoreMesh(core_axis_name="c", subcore_axis_name="s")
    out_type = [jax.ShapeDtypeStruct((NC, npad, d), jnp.float32)]
    if with_scalar:
        out_type.append(jax.ShapeDtypeStruct((NC, npad), jnp.float32))
    hh = CH // 2
    scratch = [
        pltpu.VMEM((sup, 2, 2, hh), jnp.int32),
        pltpu.VMEM((CH, d), jnp.float32),
        pltpu.VMEM((CH, d), jnp.float32),
        pltpu.VMEM_SHARED((npad, d), jnp.float32),
        pltpu.SemaphoreType.DMA,
        pltpu.SemaphoreType.DMA,
        pltpu.SemaphoreType.DMA,
        pltpu.SemaphoreType.DMA,
    ]
    if with_scalar:
        scratch += [
            pltpu.VMEM((CH,), jnp.float32),
            pltpu.VMEM((CH,), jnp.float32),
            pltpu.VMEM((nper,), jnp.float32),
            pltpu.VMEM_SHARED((npad,), jnp.float32),
        ]

    def k(*refs):
        if with_scalar:
            (y_hbm, eidx_hbm, dinv_hbm, p_hbm, sa_hbm,
             idxs, bufa, bufb, acc, gsa, gsb, ssa, ssb,
             dca, dcb, zv, sacc) = refs
        else:
            (y_hbm, eidx_hbm, p_hbm,
             idxs, bufa, bufb, acc, gsa, gsb, ssa, ssb) = refs
        c = lax.axis_index("c")
        s = lax.axis_index("s")
        bufs = (bufa, bufb)
        gsem = (gsa, gsb)
        ssem = (ssa, ssb)
        dcs = (dca, dcb) if with_scalar else (None, None)
        _fill2d(bufa, CH, 0.0)
        for i in range(nper // CH):
            pltpu.sync_copy(bufa, acc.at[pl.ds(s * nper + i * CH, CH)])
        if with_scalar:
            _fill1d(zv, nper, 0.0)
            pltpu.sync_copy(zv, sacc.at[pl.ds(s * nper, nper)])
        plsc.subcore_barrier()

        def gfire(k_):
            b = k_ & 1
            ds_ = [pltpu.async_copy(y_hbm.at[idxs.at[k_, 0, h]],
                                    bufs[b].at[pl.ds(h * hh, hh)], gsem[b])
                   for h in range(2)]
            if with_scalar:
                ds_ += [pltpu.async_copy(dinv_hbm.at[idxs.at[k_, 0, h]],
                                         dcs[b].at[pl.ds(h * hh, hh)],
                                         gsem[b])
                        for h in range(2)]
            return ds_

        def sfire(k_):
            b = k_ & 1
            ds_ = [pltpu.async_copy(bufs[b].at[pl.ds(h * hh, hh)],
                                    acc.at[idxs.at[k_, 1, h]], ssem[b],
                                    add=True)
                   for h in range(2)]
            if with_scalar:
                ds_ += [pltpu.async_copy(dcs[b].at[pl.ds(h * hh, hh)],
                                         sacc.at[idxs.at[k_, 1, h]], ssem[b],
                                         add=True)
                        for h in range(2)]
            return ds_

        def body(t, _):
            pltpu.sync_copy(eidx_hbm.at[c, s, t], idxs)
            gd = {0: gfire(0)}
            sd = {0: [], 1: []}
            for k_ in range(sup):
                b = k_ & 1
                if k_ + 1 < sup:
                    for d_ in sd[1 - b]:
                        d_.wait()
                    gd[k_ + 1] = gfire(k_ + 1)
                for d_ in gd[k_]:
                    d_.wait()
                sd[b] = sfire(k_)
            for b in (0, 1):
                for d_ in sd[b]:
                    d_.wait()
            return 0
        lax.fori_loop(0, js, body, 0)
        plsc.subcore_barrier()
        for i in range(nper // CH):
            r0 = s * nper + i * CH
            pltpu.sync_copy(acc.at[pl.ds(r0, CH)], p_hbm.at[c, pl.ds(r0, CH)])
        if with_scalar:
            pltpu.sync_copy(sacc.at[pl.ds(s * nper, nper)],
                            sa_hbm.at[c, pl.ds(s * nper, nper)])

    kfn = functools.partial(pl.kernel, mesh=mesh, out_type=out_type,
                            scratch_types=scratch)(k)
    if with_scalar:
        return kfn(yp, eidx, dinvp)
    return kfn(yp, eidx)


def _tc_scale(dinvp, xp, npad, d, bn=1024):
    g = npad // bn

    def body(dv, xr, o):
        o[...] = dv[...][:, None] * xr[...]

    return pl.pallas_call(
        body, grid=(g,),
        in_specs=[pl.BlockSpec((bn,), lambda i: (i,)),
                  pl.BlockSpec((bn, d), lambda i: (i, 0))],
        out_specs=pl.BlockSpec((bn, d), lambda i: (i, 0)),
        out_shape=jax.ShapeDtypeStruct((npad, d), jnp.float32),
    )(dinvp, xp)


def _tc_mlp(p0, p1, xp, dinvp, dinv2p, sp, W1, b1, W2, b2, npad, bn=1024):
    g = npad // bn
    d_in = xp.shape[1]
    d_h, d_out = W1.shape[0], W2.shape[0]

    def body(p0r, p1r, xr, dvr, d2r, sr, w1r, b1r, w2r, b2r, g_ref, y1_ref):
        dv = dvr[...][:, None]
        d2 = d2r[...][:, None]
        t = dv * (p0r[...] + p1r[...]) + d2 * xr[...]
        h = lax.dot_general(t, w1r[...], (((1,), (1,)), ((), ())),
                            preferred_element_type=jnp.float32)
        h = jnp.maximum(h + sr[...][:, None] * b1r[...][None, :], 0.0)
        gm = lax.dot_general(h, w2r[...], (((1,), (1,)), ((), ())),
                             preferred_element_type=jnp.float32)
        gm = gm + b2r[...][None, :]
        g_ref[...] = gm
        y1_ref[...] = dv * gm

    return pl.pallas_call(
        body, grid=(g,),
        in_specs=[pl.BlockSpec((bn, d_in), lambda i: (i, 0)),
                  pl.BlockSpec((bn, d_in), lambda i: (i, 0)),
                  pl.BlockSpec((bn, d_in), lambda i: (i, 0)),
                  pl.BlockSpec((bn,), lambda i: (i,)),
                  pl.BlockSpec((bn,), lambda i: (i,)),
                  pl.BlockSpec((bn,), lambda i: (i,)),
                  pl.BlockSpec((d_h, d_in), lambda i: (0, 0)),
                  pl.BlockSpec((d_h,), lambda i: (0,)),
                  pl.BlockSpec((d_out, d_h), lambda i: (0, 0)),
                  pl.BlockSpec((d_out,), lambda i: (0,))],
        out_specs=[pl.BlockSpec((bn, d_out), lambda i: (i, 0)),
                   pl.BlockSpec((bn, d_out), lambda i: (i, 0))],
        out_shape=[jax.ShapeDtypeStruct((npad, d_out), jnp.float32),
                   jax.ShapeDtypeStruct((npad, d_out), jnp.float32)],
    )(p0, p1, xp, dinvp, dinv2p, sp, W1, b1, W2, b2)


def _tc_combine(p0, p1, gm, dinvp, dinv2p, npad, d, bn=1024):
    g = npad // bn

    def body(p0r, p1r, gr, dvr, d2r, o):
        o[...] = (dvr[...][:, None] * (p0r[...] + p1r[...])
                  + d2r[...][:, None] * gr[...])

    return pl.pallas_call(
        body, grid=(g,),
        in_specs=[pl.BlockSpec((bn, d), lambda i: (i, 0)),
                  pl.BlockSpec((bn, d), lambda i: (i, 0)),
                  pl.BlockSpec((bn, d), lambda i: (i, 0)),
                  pl.BlockSpec((bn,), lambda i: (i,)),
                  pl.BlockSpec((bn,), lambda i: (i,))],
        out_specs=pl.BlockSpec((bn, d), lambda i: (i, 0)),
        out_shape=jax.ShapeDtypeStruct((npad, d), jnp.float32),
    )(p0, p1, gm, dinvp, dinv2p)


def kernel(x, edge_index, W1, b1, W2, b2):
    n, d_in = x.shape
    e = edge_index.shape[1]
    nw = NC * NS
    sup = 8
    assert e % nw == 0
    ept = e // nw
    js = -(-ept // (CH * sup))
    pt = js * sup * CH
    npad = -(-n // (NS * CH)) * (NS * CH)
    nper = npad // NS

    row = edge_index[0].reshape(nw, ept)
    col = edge_index[1].reshape(nw, ept)
    pad = ((0, 0), (0, pt - ept))
    colp = jnp.pad(col, pad, constant_values=n).reshape(nw, js * sup, CH)
    rowp = jnp.pad(row, pad, constant_values=n).reshape(nw, js * sup, CH)
    eidx = jnp.stack([colp, rowp], axis=2)
    eidx = eidx.reshape(NC, NS, js, sup, 2, 2, CH // 2)

    xp = jnp.pad(x, ((0, npad - n), (0, 0)))

    cnt = _sc_degree(eidx, npad, nper)
    deg = cnt[0, :n] + cnt[1, :n] + 1.0
    dinv = lax.rsqrt(deg)
    dinv2 = 1.0 / deg
    dinvp = jnp.pad(dinv, (0, npad - n))
    dinv2p = jnp.pad(dinv2, (0, npad - n))

    y0 = _tc_scale(dinvp, xp, npad, d_in)
    p, sa = _sc_propagate(y0, eidx, dinvp, npad, nper, True)
    s = dinv * (sa[0, :n] + sa[1, :n]) + dinv2
    sp = jnp.pad(s, (0, npad - n))

    g, y1 = _tc_mlp(p[0], p[1], xp, dinvp, dinv2p, sp, W1, b1, W2, b2, npad)

    p2 = _sc_propagate(y1, eidx, None, npad, nper, False)[0]
    out = _tc_combine(p2[0], p2[1], g, dinvp, dinv2p, npad, x.shape[1])
    return out[:n]


# speedup vs baseline: 26.5512x; 2.3672x over previous
import functools

import jax
import jax.numpy as jnp
from jax import lax
from jax.experimental import pallas as pl
from jax.experimental.pallas import tpu as pltpu
from jax.experimental.pallas import tpu_sc as plsc

NC = 2
NS = 16
CH = 128


def _fill1d(ref, n, val):
    def body(i, _):
        ref[pl.ds(i * 16, 16)] = jnp.full((16,), val, jnp.float32)
        return 0
    lax.fori_loop(0, n // 16, body, 0)


def _fill2d(ref, rows, val):
    def body(i, _):
        for k in range(8):
            ref[i, pl.ds(k * 16, 16)] = jnp.full((16,), val, jnp.float32)
        return 0
    lax.fori_loop(0, rows, body, 0)


def _sc_degree(eidx, npad, nper):
    js, sup = eidx.shape[2], eidx.shape[3]
    hh = CH // 2
    mesh = plsc.VectorSubcoreMesh(core_axis_name="c", subcore_axis_name="s")

    @functools.partial(
        pl.kernel, mesh=mesh,
        out_type=jax.ShapeDtypeStruct((NC, npad), jnp.float32),
        scratch_types=[
            pltpu.VMEM((sup, 2, 2, hh), jnp.int32),
            pltpu.VMEM((hh,), jnp.float32),
            pltpu.VMEM((nper,), jnp.float32),
            pltpu.VMEM_SHARED((npad,), jnp.float32),
            pltpu.SemaphoreType.DMA,
        ])
    def k(eidx_hbm, out_hbm, idxs, onesv, zv, acc, sem):
        c = lax.axis_index("c")
        s = lax.axis_index("s")
        _fill1d(onesv, hh, 1.0)
        _fill1d(zv, nper, 0.0)
        pltpu.sync_copy(zv, acc.at[pl.ds(s * nper, nper)])
        plsc.subcore_barrier()

        def body(t, _):
            pltpu.sync_copy(eidx_hbm.at[c, s, t], idxs)
            for k in range(sup):
                for h in range(2):
                    pltpu.sync_copy(onesv, acc.at[idxs.at[k, 0, h]],
                                    add=True)
            return 0
        lax.fori_loop(0, js, body, 0)
        plsc.subcore_barrier()
        pltpu.sync_copy(acc.at[pl.ds(s * nper, nper)],
                        out_hbm.at[c, pl.ds(s * nper, nper)])

    return k(eidx)


def _sc_propagate(yp, eidx, dinvp, npad, nper, with_scalar):
    js, sup = eidx.shape[2], eidx.shape[3]
    d = yp.shape[1]
    mesh = plsc.VectorSubcoreMesh(core_axis_name="c", subcore_axis_name="s")
    out_type = [jax.ShapeDtypeStruct((NC, npad, d), jnp.float32)]
    if with_scalar:
        out_type.append(jax.ShapeDtypeStruct((NC, npad), jnp.float32))
    hh = CH // 2
    scratch = [
        pltpu.VMEM((sup, 2, 2, hh), jnp.int32),
        pltpu.VMEM((CH, d), jnp.float32),
        pltpu.VMEM((CH, d), jnp.float32),
        pltpu.VMEM_SHARED((npad, d), jnp.float32),
        pltpu.SemaphoreType.DMA,
        pltpu.SemaphoreType.DMA,
        pltpu.SemaphoreType.DMA,
        pltpu.SemaphoreType.DMA,
    ]
    if with_scalar:
        scratch += [
            pltpu.VMEM((CH,), jnp.float32),
            pltpu.VMEM((CH,), jnp.float32),
            pltpu.VMEM((nper,), jnp.float32),
            pltpu.VMEM_SHARED((npad,), jnp.float32),
        ]

    def k(*refs):
        if with_scalar:
            (y_hbm, eidx_hbm, dinv_hbm, p_hbm, sa_hbm,
             idxs, bufa, bufb, acc, gsa, gsb, ssa, ssb,
             dca, dcb, zv, sacc) = refs
        else:
            (y_hbm, eidx_hbm, p_hbm,
             idxs, bufa, bufb, acc, gsa, gsb, ssa, ssb) = refs
        c = lax.axis_index("c")
        s = lax.axis_index("s")
        bufs = (bufa, bufb)
        gsem = (gsa, gsb)
        ssem = (ssa, ssb)
        dcs = (dca, dcb) if with_scalar else (None, None)
        _fill2d(bufa, CH, 0.0)
        for i in range(nper // CH):
            pltpu.sync_copy(bufa, acc.at[pl.ds(s * nper + i * CH, CH)])
        if with_scalar:
            _fill1d(zv, nper, 0.0)
            pltpu.sync_copy(zv, sacc.at[pl.ds(s * nper, nper)])
        plsc.subcore_barrier()

        def gfire(k_):
            b = k_ & 1
            ds_ = [pltpu.async_copy(y_hbm.at[idxs.at[k_, 0, h]],
                                    bufs[b].at[pl.ds(h * hh, hh)], gsem[b])
                   for h in range(2)]
            if with_scalar:
                ds_ += [pltpu.async_copy(dinv_hbm.at[idxs.at[k_, 0, h]],
                                         dcs[b].at[pl.ds(h * hh, hh)],
                                         gsem[b])
                        for h in range(2)]
            return ds_

        def sfire(k_):
            b = k_ & 1
            ds_ = [pltpu.async_copy(bufs[b].at[pl.ds(h * hh, hh)],
                                    acc.at[idxs.at[k_, 1, h]], ssem[b],
                                    add=True)
                   for h in range(2)]
            if with_scalar:
                ds_ += [pltpu.async_copy(dcs[b].at[pl.ds(h * hh, hh)],
                                         sacc.at[idxs.at[k_, 1, h]], ssem[b],
                                         add=True)
                        for h in range(2)]
            return ds_

        def body(t, _):
            pltpu.sync_copy(eidx_hbm.at[c, s, t], idxs)
            gd = {0: gfire(0)}
            sd = {0: [], 1: []}
            for k_ in range(sup):
                b = k_ & 1
                if k_ + 1 < sup:
                    for d_ in sd[1 - b]:
                        d_.wait()
                    gd[k_ + 1] = gfire(k_ + 1)
                for d_ in gd[k_]:
                    d_.wait()
                sd[b] = sfire(k_)
            for b in (0, 1):
                for d_ in sd[b]:
                    d_.wait()
            return 0
        lax.fori_loop(0, js, body, 0)
        plsc.subcore_barrier()
        for i in range(nper // CH):
            r0 = s * nper + i * CH
            pltpu.sync_copy(acc.at[pl.ds(r0, CH)], p_hbm.at[c, pl.ds(r0, CH)])
        if with_scalar:
            pltpu.sync_copy(sacc.at[pl.ds(s * nper, nper)],
                            sa_hbm.at[c, pl.ds(s * nper, nper)])

    kfn = functools.partial(pl.kernel, mesh=mesh, out_type=out_type,
                            scratch_types=scratch)(k)
    if with_scalar:
        return kfn(yp, eidx, dinvp)
    return kfn(yp, eidx)


def _tc_scale(dinvp, xp, npad, d, bn=1024):
    g = npad // bn

    def body(dv, xr, o):
        o[...] = dv[...][:, None] * xr[...]

    return pl.pallas_call(
        body, grid=(g,),
        in_specs=[pl.BlockSpec((bn,), lambda i: (i,)),
                  pl.BlockSpec((bn, d), lambda i: (i, 0))],
        out_specs=pl.BlockSpec((bn, d), lambda i: (i, 0)),
        out_shape=jax.ShapeDtypeStruct((npad, d), jnp.float32),
    )(dinvp, xp)


def _tc_mlp(p0, p1, xp, dinvp, dinv2p, sp, W1, b1, W2, b2, npad, bn=1024):
    g = npad // bn
    d_in = xp.shape[1]
    d_h, d_out = W1.shape[0], W2.shape[0]

    def body(p0r, p1r, xr, dvr, d2r, sr, w1r, b1r, w2r, b2r, g_ref, y1_ref):
        dv = dvr[...][:, None]
        d2 = d2r[...][:, None]
        t = dv * (p0r[...] + p1r[...]) + d2 * xr[...]
        h = lax.dot_general(t, w1r[...], (((1,), (1,)), ((), ())),
                            preferred_element_type=jnp.float32)
        h = jnp.maximum(h + sr[...][:, None] * b1r[...][None, :], 0.0)
        gm = lax.dot_general(h, w2r[...], (((1,), (1,)), ((), ())),
                             preferred_element_type=jnp.float32)
        gm = gm + b2r[...][None, :]
        g_ref[...] = gm
        y1_ref[...] = dv * gm

    return pl.pallas_call(
        body, grid=(g,),
        in_specs=[pl.BlockSpec((bn, d_in), lambda i: (i, 0)),
                  pl.BlockSpec((bn, d_in), lambda i: (i, 0)),
                  pl.BlockSpec((bn, d_in), lambda i: (i, 0)),
                  pl.BlockSpec((bn,), lambda i: (i,)),
                  pl.BlockSpec((bn,), lambda i: (i,)),
                  pl.BlockSpec((bn,), lambda i: (i,)),
                  pl.BlockSpec((d_h, d_in), lambda i: (0, 0)),
                  pl.BlockSpec((d_h,), lambda i: (0,)),
                  pl.BlockSpec((d_out, d_h), lambda i: (0, 0)),
                  pl.BlockSpec((d_out,), lambda i: (0,))],
        out_specs=[pl.BlockSpec((bn, d_out), lambda i: (i, 0)),
                   pl.BlockSpec((bn, d_out), lambda i: (i, 0))],
        out_shape=[jax.ShapeDtypeStruct((npad, d_out), jnp.float32),
                   jax.ShapeDtypeStruct((npad, d_out), jnp.float32)],
    )(p0, p1, xp, dinvp, dinv2p, sp, W1, b1, W2, b2)


def _tc_combine(p0, p1, gm, dinvp, dinv2p, npad, d, bn=1024):
    g = npad // bn

    def body(p0r, p1r, gr, dvr, d2r, o):
        o[...] = (dvr[...][:, None] * (p0r[...] + p1r[...])
                  + d2r[...][:, None] * gr[...])

    return pl.pallas_call(
        body, grid=(g,),
        in_specs=[pl.BlockSpec((bn, d), lambda i: (i, 0)),
                  pl.BlockSpec((bn, d), lambda i: (i, 0)),
                  pl.BlockSpec((bn, d), lambda i: (i, 0)),
                  pl.BlockSpec((bn,), lambda i: (i,)),
                  pl.BlockSpec((bn,), lambda i: (i,))],
        out_specs=pl.BlockSpec((bn, d), lambda i: (i, 0)),
        out_shape=jax.ShapeDtypeStruct((npad, d), jnp.float32),
    )(p0, p1, gm, dinvp, dinv2p)


def kernel(x, edge_index, W1, b1, W2, b2):
    n, d_in = x.shape
    e = edge_index.shape[1]
    nw = NC * NS
    sup = 8
    assert e % nw == 0
    ept = e // nw
    js = -(-ept // (CH * sup))
    pt = js * sup * CH
    npad = -(-n // (NS * CH)) * (NS * CH)
    nper = npad // NS

    row = edge_index[0].reshape(nw, ept)
    col = edge_index[1].reshape(nw, ept)
    sink = n + (jnp.arange(pt - ept, dtype=jnp.int32) % (npad - n))
    pad_blk = jnp.broadcast_to(sink, (nw, pt - ept))
    colp = jnp.concatenate([col, pad_blk], axis=1).reshape(nw, js * sup, CH)
    rowp = jnp.concatenate([row, pad_blk], axis=1).reshape(nw, js * sup, CH)
    eidx = jnp.stack([colp, rowp], axis=2)
    eidx = eidx.reshape(NC, NS, js, sup, 2, 2, CH // 2)

    xp = jnp.pad(x, ((0, npad - n), (0, 0)))

    cnt = _sc_degree(eidx, npad, nper)
    deg = cnt[0, :n] + cnt[1, :n] + 1.0
    dinv = lax.rsqrt(deg)
    dinv2 = 1.0 / deg
    dinvp = jnp.pad(dinv, (0, npad - n))
    dinv2p = jnp.pad(dinv2, (0, npad - n))

    y0 = _tc_scale(dinvp, xp, npad, d_in)
    p, sa = _sc_propagate(y0, eidx, dinvp, npad, nper, True)
    s = dinv * (sa[0, :n] + sa[1, :n]) + dinv2
    sp = jnp.pad(s, (0, npad - n))

    g, y1 = _tc_mlp(p[0], p[1], xp, dinvp, dinv2p, sp, W1, b1, W2, b2, npad)

    p2 = _sc_propagate(y1, eidx, None, npad, nper, False)[0]
    out = _tc_combine(p2[0], p2[1], g, dinvp, dinv2p, npad, x.shape[1])
    return out[:n]


# speedup vs baseline: 28.2395x; 1.0636x over previous
import functools

import jax
import jax.numpy as jnp
from jax import lax
from jax.experimental import pallas as pl
from jax.experimental.pallas import tpu as pltpu
from jax.experimental.pallas import tpu_sc as plsc

NC = 2
NS = 16
CH = 96
SUP = 7
NBUF = 3


def _fill1d(ref, n, val):
    def body(i, _):
        ref[pl.ds(i * 16, 16)] = jnp.full((16,), val, jnp.float32)
        return 0
    lax.fori_loop(0, n // 16, body, 0)


def _fill2d(ref, rows, val):
    def body(i, _):
        for k in range(8):
            ref[i, pl.ds(k * 16, 16)] = jnp.full((16,), val, jnp.float32)
        return 0
    lax.fori_loop(0, rows, body, 0)


def _sc_degree(eidx, npad, nper):
    js, sup = eidx.shape[2], eidx.shape[3]
    mesh = plsc.VectorSubcoreMesh(core_axis_name="c", subcore_axis_name="s")

    @functools.partial(
        pl.kernel, mesh=mesh,
        out_type=jax.ShapeDtypeStruct((NC, npad), jnp.float32),
        scratch_types=[
            pltpu.VMEM((sup, 2, CH), jnp.int32),
            pltpu.VMEM((CH,), jnp.float32),
            pltpu.VMEM((nper,), jnp.float32),
            pltpu.VMEM_SHARED((npad,), jnp.float32),
            pltpu.SemaphoreType.DMA,
        ])
    def k(eidx_hbm, out_hbm, idxs, onesv, zv, acc, sem):
        c = lax.axis_index("c")
        s = lax.axis_index("s")
        _fill1d(onesv, CH, 1.0)
        _fill1d(zv, nper, 0.0)
        pltpu.sync_copy(zv, acc.at[pl.ds(s * nper, nper)])
        plsc.subcore_barrier()

        def body(t, _):
            pltpu.sync_copy(eidx_hbm.at[c, s, t], idxs)
            for k in range(sup):
                pltpu.sync_copy(onesv, acc.at[idxs.at[k, 0]], add=True)
            return 0
        lax.fori_loop(0, js, body, 0)
        plsc.subcore_barrier()
        pltpu.sync_copy(acc.at[pl.ds(s * nper, nper)],
                        out_hbm.at[c, pl.ds(s * nper, nper)])

    return k(eidx)


def _sc_propagate(yp, eidx, dinvp, npad, nper, with_scalar):
    js, sup = eidx.shape[2], eidx.shape[3]
    d = yp.shape[1]
    mesh = plsc.VectorSubcoreMesh(core_axis_name="c", subcore_axis_name="s")
    out_type = [jax.ShapeDtypeStruct((NC, npad, d), jnp.float32)]
    if with_scalar:
        out_type.append(jax.ShapeDtypeStruct((NC, npad), jnp.float32))
    scratch = [
        pltpu.VMEM((sup, 2, CH), jnp.int32),
    ]
    scratch += [pltpu.VMEM((CH, d), jnp.float32) for _ in range(NBUF)]
    scratch += [pltpu.VMEM_SHARED((npad, d), jnp.float32)]
    scratch += [pltpu.SemaphoreType.DMA for _ in range(2 * NBUF)]
    if with_scalar:
        scratch += [
            pltpu.VMEM((NBUF, CH), jnp.float32),
            pltpu.VMEM((nper,), jnp.float32),
            pltpu.VMEM_SHARED((npad,), jnp.float32),
        ]

    def k(*refs):
        if with_scalar:
            (y_hbm, eidx_hbm, dinv_hbm, p_hbm, sa_hbm, idxs, *rest) = refs
            bufs = rest[:NBUF]
            acc = rest[NBUF]
            gsem = rest[NBUF + 1:2 * NBUF + 1]
            ssem = rest[2 * NBUF + 1:3 * NBUF + 1]
            dcs, zv, sacc = rest[3 * NBUF + 1:]
        else:
            (y_hbm, eidx_hbm, p_hbm, idxs, *rest) = refs
            bufs = rest[:NBUF]
            acc = rest[NBUF]
            gsem = rest[NBUF + 1:2 * NBUF + 1]
            ssem = rest[2 * NBUF + 1:3 * NBUF + 1]
        c = lax.axis_index("c")
        s = lax.axis_index("s")
        _fill2d(bufs[0], CH, 0.0)
        for i in range(nper // CH):
            pltpu.sync_copy(bufs[0], acc.at[pl.ds(s * nper + i * CH, CH)])
        if nper % CH:
            pltpu.sync_copy(
                bufs[0].at[pl.ds(0, nper % CH)],
                acc.at[pl.ds(s * nper + (nper // CH) * CH, nper % CH)])
        if with_scalar:
            _fill1d(zv, nper, 0.0)
            pltpu.sync_copy(zv, sacc.at[pl.ds(s * nper, nper)])
        plsc.subcore_barrier()

        def gfire(k_):
            b = k_ % NBUF
            ds_ = [pltpu.async_copy(y_hbm.at[idxs.at[k_, 0]], bufs[b],
                                    gsem[b])]
            if with_scalar:
                ds_.append(pltpu.async_copy(dinv_hbm.at[idxs.at[k_, 0]],
                                            dcs.at[b], gsem[b]))
            return ds_

        def sfire(k_):
            b = k_ % NBUF
            ds_ = [pltpu.async_copy(bufs[b], acc.at[idxs.at[k_, 1]],
                                    ssem[b], add=True)]
            if with_scalar:
                ds_.append(pltpu.async_copy(dcs.at[b],
                                            sacc.at[idxs.at[k_, 1]],
                                            ssem[b], add=True))
            return ds_

        def body(t, _):
            pltpu.sync_copy(eidx_hbm.at[c, s, t], idxs)
            gd = {0: gfire(0)}
            if sup > 1:
                gd[1] = gfire(1)
            sd = {b: [] for b in range(NBUF)}
            for k_ in range(sup):
                b = k_ % NBUF
                if k_ + 2 < sup:
                    for d_ in sd[(k_ + 2) % NBUF]:
                        d_.wait()
                    sd[(k_ + 2) % NBUF] = []
                    gd[k_ + 2] = gfire(k_ + 2)
                for d_ in gd[k_]:
                    d_.wait()
                sd[b] = sfire(k_)
            for b in range(NBUF):
                for d_ in sd[b]:
                    d_.wait()
            return 0
        lax.fori_loop(0, js, body, 0)
        plsc.subcore_barrier()
        for i in range(nper // CH):
            r0 = s * nper + i * CH
            pltpu.sync_copy(acc.at[pl.ds(r0, CH)], p_hbm.at[c, pl.ds(r0, CH)])
        if nper % CH:
            r0 = s * nper + (nper // CH) * CH
            pltpu.sync_copy(acc.at[pl.ds(r0, nper % CH)],
                            p_hbm.at[c, pl.ds(r0, nper % CH)])
        if with_scalar:
            pltpu.sync_copy(sacc.at[pl.ds(s * nper, nper)],
                            sa_hbm.at[c, pl.ds(s * nper, nper)])

    kfn = functools.partial(pl.kernel, mesh=mesh, out_type=out_type,
                            scratch_types=scratch)(k)
    if with_scalar:
        return kfn(yp, eidx, dinvp)
    return kfn(yp, eidx)


def _tc_scale(dinvp, xp, npad, d, bn=1024):
    g = npad // bn

    def body(dv, xr, o):
        o[...] = dv[...][:, None] * xr[...]

    return pl.pallas_call(
        body, grid=(g,),
        in_specs=[pl.BlockSpec((bn,), lambda i: (i,)),
                  pl.BlockSpec((bn, d), lambda i: (i, 0))],
        out_specs=pl.BlockSpec((bn, d), lambda i: (i, 0)),
        out_shape=jax.ShapeDtypeStruct((npad, d), jnp.float32),
    )(dinvp, xp)


def _tc_mlp(p0, p1, xp, dinvp, dinv2p, sp, W1, b1, W2, b2, npad, bn=1024):
    g = npad // bn
    d_in = xp.shape[1]
    d_h, d_out = W1.shape[0], W2.shape[0]

    def body(p0r, p1r, xr, dvr, d2r, sr, w1r, b1r, w2r, b2r, g_ref, y1_ref):
        dv = dvr[...][:, None]
        d2 = d2r[...][:, None]
        t = dv * (p0r[...] + p1r[...]) + d2 * xr[...]
        h = lax.dot_general(t, w1r[...], (((1,), (1,)), ((), ())),
                            preferred_element_type=jnp.float32)
        h = jnp.maximum(h + sr[...][:, None] * b1r[...][None, :], 0.0)
        gm = lax.dot_general(h, w2r[...], (((1,), (1,)), ((), ())),
                             preferred_element_type=jnp.float32)
        gm = gm + b2r[...][None, :]
        g_ref[...] = gm
        y1_ref[...] = dv * gm

    return pl.pallas_call(
        body, grid=(g,),
        in_specs=[pl.BlockSpec((bn, d_in), lambda i: (i, 0)),
                  pl.BlockSpec((bn, d_in), lambda i: (i, 0)),
                  pl.BlockSpec((bn, d_in), lambda i: (i, 0)),
                  pl.BlockSpec((bn,), lambda i: (i,)),
                  pl.BlockSpec((bn,), lambda i: (i,)),
                  pl.BlockSpec((bn,), lambda i: (i,)),
                  pl.BlockSpec((d_h, d_in), lambda i: (0, 0)),
                  pl.BlockSpec((d_h,), lambda i: (0,)),
                  pl.BlockSpec((d_out, d_h), lambda i: (0, 0)),
                  pl.BlockSpec((d_out,), lambda i: (0,))],
        out_specs=[pl.BlockSpec((bn, d_out), lambda i: (i, 0)),
                   pl.BlockSpec((bn, d_out), lambda i: (i, 0))],
        out_shape=[jax.ShapeDtypeStruct((npad, d_out), jnp.float32),
                   jax.ShapeDtypeStruct((npad, d_out), jnp.float32)],
    )(p0, p1, xp, dinvp, dinv2p, sp, W1, b1, W2, b2)


def _tc_combine(p0, p1, gm, dinvp, dinv2p, npad, d, bn=1024):
    g = npad // bn

    def body(p0r, p1r, gr, dvr, d2r, o):
        o[...] = (dvr[...][:, None] * (p0r[...] + p1r[...])
                  + d2r[...][:, None] * gr[...])

    return pl.pallas_call(
        body, grid=(g,),
        in_specs=[pl.BlockSpec((bn, d), lambda i: (i, 0)),
                  pl.BlockSpec((bn, d), lambda i: (i, 0)),
                  pl.BlockSpec((bn, d), lambda i: (i, 0)),
                  pl.BlockSpec((bn,), lambda i: (i,)),
                  pl.BlockSpec((bn,), lambda i: (i,))],
        out_specs=pl.BlockSpec((bn, d), lambda i: (i, 0)),
        out_shape=jax.ShapeDtypeStruct((npad, d), jnp.float32),
    )(p0, p1, gm, dinvp, dinv2p)


def kernel(x, edge_index, W1, b1, W2, b2):
    n, d_in = x.shape
    e = edge_index.shape[1]
    nw = NC * NS
    sup = SUP
    assert e % nw == 0
    ept = e // nw
    js = -(-ept // (CH * sup))
    pt = js * sup * CH
    npad = -(-(n + 1) // 2048) * 2048
    nper = npad // NS

    row = edge_index[0].reshape(nw, ept)
    col = edge_index[1].reshape(nw, ept)
    sink = n + (jnp.arange(pt - ept, dtype=jnp.int32) % (npad - n))
    pad_blk = jnp.broadcast_to(sink, (nw, pt - ept))
    colp = jnp.concatenate([col, pad_blk], axis=1).reshape(nw, js * sup, CH)
    rowp = jnp.concatenate([row, pad_blk], axis=1).reshape(nw, js * sup, CH)
    eidx = jnp.stack([colp, rowp], axis=2)
    eidx = eidx.reshape(NC, NS, js, sup, 2, CH)

    xp = jnp.pad(x, ((0, npad - n), (0, 0)))

    cnt = _sc_degree(eidx, npad, nper)
    deg = cnt[0, :n] + cnt[1, :n] + 1.0
    dinv = lax.rsqrt(deg)
    dinv2 = 1.0 / deg
    dinvp = jnp.pad(dinv, (0, npad - n))
    dinv2p = jnp.pad(dinv2, (0, npad - n))

    y0 = _tc_scale(dinvp, xp, npad, d_in)
    p, sa = _sc_propagate(y0, eidx, dinvp, npad, nper, True)
    s = dinv * (sa[0, :n] + sa[1, :n]) + dinv2
    sp = jnp.pad(s, (0, npad - n))

    g, y1 = _tc_mlp(p[0], p[1], xp, dinvp, dinv2p, sp, W1, b1, W2, b2, npad)

    p2 = _sc_propagate(y1, eidx, None, npad, nper, False)[0]
    out = _tc_combine(p2[0], p2[1], g, dinvp, dinv2p, npad, x.shape[1])
    return out[:n]


# speedup vs baseline: 33.7235x; 1.1942x over previous
import functools

import jax
import jax.numpy as jnp
from jax import lax
from jax.experimental import pallas as pl
from jax.experimental.pallas import tpu as pltpu
from jax.experimental.pallas import tpu_sc as plsc

NC = 2
NS = 16
CH = 80
SUP = 25
NBUF = 4


def _fill1d(ref, n, val):
    def body(i, _):
        ref[pl.ds(i * 16, 16)] = jnp.full((16,), val, jnp.float32)
        return 0
    lax.fori_loop(0, n // 16, body, 0)


def _fill2d(ref, rows, val):
    def body(i, _):
        for k in range(8):
            ref[i, pl.ds(k * 16, 16)] = jnp.full((16,), val, jnp.float32)
        return 0
    lax.fori_loop(0, rows, body, 0)


def _sc_degree(er, npad, nper):
    js, sup = er.shape[3], er.shape[4]
    mesh = plsc.VectorSubcoreMesh(core_axis_name="c", subcore_axis_name="s")

    @functools.partial(
        pl.kernel, mesh=mesh,
        out_type=jax.ShapeDtypeStruct((NC, npad), jnp.float32),
        scratch_types=[
            pltpu.VMEM((sup, CH), jnp.int32),
            pltpu.VMEM((CH,), jnp.float32),
            pltpu.VMEM((nper,), jnp.float32),
            pltpu.VMEM_SHARED((npad,), jnp.float32),
            pltpu.SemaphoreType.DMA,
        ])
    def k(er_hbm, out_hbm, idxs, onesv, zv, acc, sem):
        c = lax.axis_index("c")
        s = lax.axis_index("s")
        _fill1d(onesv, CH, 1.0)
        _fill1d(zv, nper, 0.0)
        pltpu.sync_copy(zv, acc.at[pl.ds(s * nper, nper)])
        plsc.subcore_barrier()

        def body(t, _):
            pltpu.sync_copy(er_hbm.at[1, c, s, t], idxs)
            for k in range(sup):
                pltpu.sync_copy(onesv, acc.at[idxs.at[k]], add=True)
            return 0
        lax.fori_loop(0, js, body, 0)
        plsc.subcore_barrier()
        pltpu.sync_copy(acc.at[pl.ds(s * nper, nper)],
                        out_hbm.at[c, pl.ds(s * nper, nper)])

    return k(er)


def _sc_propagate(yp, er, dinvp, npad, nper, with_scalar):
    js, sup = er.shape[3], er.shape[4]
    d = yp.shape[1]
    mesh = plsc.VectorSubcoreMesh(core_axis_name="c", subcore_axis_name="s")
    out_type = [jax.ShapeDtypeStruct((NC, npad, d), jnp.float32)]
    if with_scalar:
        out_type.append(jax.ShapeDtypeStruct((NC, npad), jnp.float32))
    nbuf = NBUF - 1 if with_scalar else NBUF
    scratch = [
        pltpu.VMEM((sup, CH), jnp.int32),
        pltpu.VMEM((sup, CH), jnp.int32),
    ]
    scratch += [pltpu.VMEM((CH, d), jnp.float32) for _ in range(nbuf)]
    scratch += [pltpu.VMEM_SHARED((npad, d), jnp.float32)]
    scratch += [pltpu.SemaphoreType.DMA for _ in range(2 * nbuf)]
    if with_scalar:
        scratch += [
            pltpu.VMEM((nbuf, CH), jnp.float32),
            pltpu.VMEM_SHARED((npad,), jnp.float32),
        ]

    def k(*refs):
        if with_scalar:
            (y_hbm, er_hbm, dinv_hbm, p_hbm, sa_hbm, idxc, idxr,
             *rest) = refs
        else:
            (y_hbm, er_hbm, p_hbm, idxc, idxr, *rest) = refs
        bufs = rest[:nbuf]
        acc = rest[nbuf]
        gsem = rest[nbuf + 1:2 * nbuf + 1]
        ssem = rest[2 * nbuf + 1:3 * nbuf + 1]
        if with_scalar:
            dcs, sacc = rest[3 * nbuf + 1:]
        c = lax.axis_index("c")
        s = lax.axis_index("s")
        _fill2d(bufs[0], CH, 0.0)
        for i in range(nper // CH):
            pltpu.sync_copy(bufs[0], acc.at[pl.ds(s * nper + i * CH, CH)])
        if nper % CH:
            pltpu.sync_copy(
                bufs[0].at[pl.ds(0, nper % CH)],
                acc.at[pl.ds(s * nper + (nper // CH) * CH, nper % CH)])
        if with_scalar:
            for i in range(nper // 128):
                pltpu.sync_copy(bufs[0].at[0],
                                sacc.at[pl.ds(s * nper + i * 128, 128)])
        plsc.subcore_barrier()

        def gfire(k_):
            b = k_ % nbuf
            ds_ = [pltpu.async_copy(y_hbm.at[idxc.at[k_]], bufs[b],
                                    gsem[b])]
            if with_scalar:
                ds_.append(pltpu.async_copy(dinv_hbm.at[idxc.at[k_]],
                                            dcs.at[b], gsem[b]))
            return ds_

        def sfire(k_):
            b = k_ % nbuf
            ds_ = [pltpu.async_copy(bufs[b], acc.at[idxr.at[k_]],
                                    ssem[b], add=True)]
            if with_scalar:
                ds_.append(pltpu.async_copy(dcs.at[b],
                                            sacc.at[idxr.at[k_]],
                                            ssem[b], add=True))
            return ds_

        def body(t, _):
            pltpu.sync_copy(er_hbm.at[1, c, s, t], idxc)
            pltpu.sync_copy(er_hbm.at[0, c, s, t], idxr)
            depth = nbuf - 1
            gd = {}
            for j in range(min(depth, sup)):
                gd[j] = gfire(j)
            sd = {b: [] for b in range(nbuf)}
            for k_ in range(sup):
                b = k_ % nbuf
                if k_ + depth < sup:
                    for d_ in sd[(k_ + depth) % nbuf]:
                        d_.wait()
                    sd[(k_ + depth) % nbuf] = []
                    gd[k_ + depth] = gfire(k_ + depth)
                for d_ in gd[k_]:
                    d_.wait()
                sd[b] = sfire(k_)
            for b in range(nbuf):
                for d_ in sd[b]:
                    d_.wait()
            return 0
        lax.fori_loop(0, js, body, 0)
        plsc.subcore_barrier()
        for i in range(nper // CH):
            r0 = s * nper + i * CH
            pltpu.sync_copy(acc.at[pl.ds(r0, CH)], p_hbm.at[c, pl.ds(r0, CH)])
        if nper % CH:
            r0 = s * nper + (nper // CH) * CH
            pltpu.sync_copy(acc.at[pl.ds(r0, nper % CH)],
                            p_hbm.at[c, pl.ds(r0, nper % CH)])
        if with_scalar:
            pltpu.sync_copy(sacc.at[pl.ds(s * nper, nper)],
                            sa_hbm.at[c, pl.ds(s * nper, nper)])

    kfn = functools.partial(pl.kernel, mesh=mesh, out_type=out_type,
                            scratch_types=scratch)(k)
    if with_scalar:
        return kfn(yp, er, dinvp)
    return kfn(yp, er)


def _tc_scale(dinvp, xp, npad, d, bn=1024):
    g = npad // bn

    def body(dv, xr, o):
        o[...] = dv[...][:, None] * xr[...]

    return pl.pallas_call(
        body, grid=(g,),
        in_specs=[pl.BlockSpec((bn,), lambda i: (i,)),
                  pl.BlockSpec((bn, d), lambda i: (i, 0))],
        out_specs=pl.BlockSpec((bn, d), lambda i: (i, 0)),
        out_shape=jax.ShapeDtypeStruct((npad, d), jnp.float32),
    )(dinvp, xp)


def _tc_mlp(p0, p1, xp, dinvp, dinv2p, sp, W1, b1, W2, b2, npad, bn=1024):
    g = npad // bn
    d_in = xp.shape[1]
    d_h, d_out = W1.shape[0], W2.shape[0]

    def body(p0r, p1r, xr, dvr, d2r, sr, w1r, b1r, w2r, b2r, g_ref, y1_ref):
        dv = dvr[...][:, None]
        d2 = d2r[...][:, None]
        t = dv * (p0r[...] + p1r[...]) + d2 * xr[...]
        h = lax.dot_general(t, w1r[...], (((1,), (1,)), ((), ())),
                            preferred_element_type=jnp.float32)
        h = jnp.maximum(h + sr[...][:, None] * b1r[...][None, :], 0.0)
        gm = lax.dot_general(h, w2r[...], (((1,), (1,)), ((), ())),
                             preferred_element_type=jnp.float32)
        gm = gm + b2r[...][None, :]
        g_ref[...] = gm
        y1_ref[...] = dv * gm

    return pl.pallas_call(
        body, grid=(g,),
        in_specs=[pl.BlockSpec((bn, d_in), lambda i: (i, 0)),
                  pl.BlockSpec((bn, d_in), lambda i: (i, 0)),
                  pl.BlockSpec((bn, d_in), lambda i: (i, 0)),
                  pl.BlockSpec((bn,), lambda i: (i,)),
                  pl.BlockSpec((bn,), lambda i: (i,)),
                  pl.BlockSpec((bn,), lambda i: (i,)),
                  pl.BlockSpec((d_h, d_in), lambda i: (0, 0)),
                  pl.BlockSpec((d_h,), lambda i: (0,)),
                  pl.BlockSpec((d_out, d_h), lambda i: (0, 0)),
                  pl.BlockSpec((d_out,), lambda i: (0,))],
        out_specs=[pl.BlockSpec((bn, d_out), lambda i: (i, 0)),
                   pl.BlockSpec((bn, d_out), lambda i: (i, 0))],
        out_shape=[jax.ShapeDtypeStruct((npad, d_out), jnp.float32),
                   jax.ShapeDtypeStruct((npad, d_out), jnp.float32)],
    )(p0, p1, xp, dinvp, dinv2p, sp, W1, b1, W2, b2)


def _tc_combine(p0, p1, gm, dinvp, dinv2p, npad, d, bn=1024):
    g = npad // bn

    def body(p0r, p1r, gr, dvr, d2r, o):
        o[...] = (dvr[...][:, None] * (p0r[...] + p1r[...])
                  + d2r[...][:, None] * gr[...])

    return pl.pallas_call(
        body, grid=(g,),
        in_specs=[pl.BlockSpec((bn, d), lambda i: (i, 0)),
                  pl.BlockSpec((bn, d), lambda i: (i, 0)),
                  pl.BlockSpec((bn, d), lambda i: (i, 0)),
                  pl.BlockSpec((bn,), lambda i: (i,)),
                  pl.BlockSpec((bn,), lambda i: (i,))],
        out_specs=pl.BlockSpec((bn, d), lambda i: (i, 0)),
        out_shape=jax.ShapeDtypeStruct((npad, d), jnp.float32),
    )(p0, p1, gm, dinvp, dinv2p)


def kernel(x, edge_index, W1, b1, W2, b2):
    n, d_in = x.shape
    e = edge_index.shape[1]
    nw = NC * NS
    sup = SUP
    assert e % (nw * CH * sup) == 0, "edge count must tile evenly"
    ept = e // nw
    js = ept // (CH * sup)
    npad = -(-(n + 1) // 2048) * 2048
    nper = npad // NS

    er = edge_index.reshape(2, NC, NS, js, sup, CH)

    xp = jnp.pad(x, ((0, npad - n), (0, 0)))

    cnt = _sc_degree(er, npad, nper)
    deg = cnt[0, :n] + cnt[1, :n] + 1.0
    dinv = lax.rsqrt(deg)
    dinv2 = 1.0 / deg
    dinvp = jnp.pad(dinv, (0, npad - n))
    dinv2p = jnp.pad(dinv2, (0, npad - n))

    y0 = _tc_scale(dinvp, xp, npad, d_in)
    p, sa = _sc_propagate(y0, er, dinvp, npad, nper, True)
    s = dinv * (sa[0, :n] + sa[1, :n]) + dinv2
    sp = jnp.pad(s, (0, npad - n))

    g, y1 = _tc_mlp(p[0], p[1], xp, dinvp, dinv2p, sp, W1, b1, W2, b2, npad)

    p2 = _sc_propagate(y1, er, None, npad, nper, False)[0]
    out = _tc_combine(p2[0], p2[1], g, dinvp, dinv2p, npad, x.shape[1])
    return out[:n]


# speedup vs baseline: 34.3751x; 1.0193x over previous
import functools

import jax
import jax.numpy as jnp
from jax import lax
from jax.experimental import pallas as pl
from jax.experimental.pallas import tpu as pltpu
from jax.experimental.pallas import tpu_sc as plsc

NC = 2
NS = 16
CH = 80
SUP = 25
NBUF = 4


def _fill1d(ref, n, val):
    def body(i, _):
        ref[pl.ds(i * 16, 16)] = jnp.full((16,), val, jnp.float32)
        return 0
    lax.fori_loop(0, n // 16, body, 0)


def _fill2d(ref, rows, val):
    def body(i, _):
        for k in range(8):
            ref[i, pl.ds(k * 16, 16)] = jnp.full((16,), val, jnp.float32)
        return 0
    lax.fori_loop(0, rows, body, 0)


def _sc_degree(er, npad, nper):
    js, sup = er.shape[3], er.shape[4]
    mesh = plsc.VectorSubcoreMesh(core_axis_name="c", subcore_axis_name="s")

    @functools.partial(
        pl.kernel, mesh=mesh,
        out_type=jax.ShapeDtypeStruct((NC, npad), jnp.float32),
        scratch_types=[
            pltpu.VMEM((sup, CH), jnp.int32),
            pltpu.VMEM((CH,), jnp.float32),
            pltpu.VMEM((nper,), jnp.float32),
            pltpu.VMEM_SHARED((npad,), jnp.float32),
            pltpu.SemaphoreType.DMA,
            pltpu.SemaphoreType.DMA,
            pltpu.SemaphoreType.DMA,
            pltpu.SemaphoreType.DMA,
        ])
    def k(er_hbm, out_hbm, idxs, onesv, zv, acc, *sems):
        c = lax.axis_index("c")
        s = lax.axis_index("s")
        _fill1d(onesv, CH, 1.0)
        _fill1d(zv, nper, 0.0)
        pltpu.sync_copy(zv, acc.at[pl.ds(s * nper, nper)])
        plsc.subcore_barrier()

        def body(t, _):
            pltpu.sync_copy(er_hbm.at[1, c, s, t], idxs)
            ds = {}
            for k in range(sup):
                if k >= 4:
                    ds[k - 4].wait()
                ds[k] = pltpu.async_copy(onesv, acc.at[idxs.at[k]],
                                         sems[k % 4], add=True)
            for k in range(max(0, sup - 4), sup):
                ds[k].wait()
            return 0
        lax.fori_loop(0, js, body, 0)
        plsc.subcore_barrier()
        pltpu.sync_copy(acc.at[pl.ds(s * nper, nper)],
                        out_hbm.at[c, pl.ds(s * nper, nper)])

    return k(er)


def _sc_propagate(yp, er, dinvp, npad, nper, with_scalar):
    js, sup = er.shape[3], er.shape[4]
    d = yp.shape[1]
    mesh = plsc.VectorSubcoreMesh(core_axis_name="c", subcore_axis_name="s")
    out_type = [jax.ShapeDtypeStruct((NC, npad, d), jnp.float32)]
    if with_scalar:
        out_type.append(jax.ShapeDtypeStruct((NC, npad), jnp.float32))
    nbuf = NBUF - 1 if with_scalar else NBUF
    scratch = [
        pltpu.VMEM((sup, CH), jnp.int32),
        pltpu.VMEM((sup, CH), jnp.int32),
    ]
    scratch += [pltpu.VMEM((CH, d), jnp.float32) for _ in range(nbuf)]
    scratch += [pltpu.VMEM_SHARED((npad, d), jnp.float32)]
    scratch += [pltpu.SemaphoreType.DMA for _ in range(2 * nbuf)]
    if with_scalar:
        scratch += [
            pltpu.VMEM((nbuf, CH), jnp.float32),
            pltpu.VMEM_SHARED((npad,), jnp.float32),
        ]

    def k(*refs):
        if with_scalar:
            (y_hbm, er_hbm, dinv_hbm, p_hbm, sa_hbm, idxc, idxr,
             *rest) = refs
        else:
            (y_hbm, er_hbm, p_hbm, idxc, idxr, *rest) = refs
        bufs = rest[:nbuf]
        acc = rest[nbuf]
        gsem = rest[nbuf + 1:2 * nbuf + 1]
        ssem = rest[2 * nbuf + 1:3 * nbuf + 1]
        if with_scalar:
            dcs, sacc = rest[3 * nbuf + 1:]
        c = lax.axis_index("c")
        s = lax.axis_index("s")
        _fill2d(bufs[0], CH, 0.0)
        for i in range(nper // CH):
            pltpu.sync_copy(bufs[0], acc.at[pl.ds(s * nper + i * CH, CH)])
        if nper % CH:
            pltpu.sync_copy(
                bufs[0].at[pl.ds(0, nper % CH)],
                acc.at[pl.ds(s * nper + (nper // CH) * CH, nper % CH)])
        if with_scalar:
            for i in range(nper // 128):
                pltpu.sync_copy(bufs[0].at[0],
                                sacc.at[pl.ds(s * nper + i * 128, 128)])
        plsc.subcore_barrier()

        def gfire(k_):
            b = k_ % nbuf
            ds_ = [pltpu.async_copy(y_hbm.at[idxc.at[k_]], bufs[b],
                                    gsem[b])]
            if with_scalar:
                ds_.append(pltpu.async_copy(dinv_hbm.at[idxc.at[k_]],
                                            dcs.at[b], gsem[b]))
            return ds_

        def sfire(k_):
            b = k_ % nbuf
            ds_ = [pltpu.async_copy(bufs[b], acc.at[idxr.at[k_]],
                                    ssem[b], add=True)]
            if with_scalar:
                ds_.append(pltpu.async_copy(dcs.at[b],
                                            sacc.at[idxr.at[k_]],
                                            ssem[b], add=True))
            return ds_

        def body(t, _):
            pltpu.sync_copy(er_hbm.at[1, c, s, t], idxc)
            pltpu.sync_copy(er_hbm.at[0, c, s, t], idxr)
            depth = nbuf - 1
            gd = {}
            for j in range(min(depth, sup)):
                gd[j] = gfire(j)
            sd = {b: [] for b in range(nbuf)}
            for k_ in range(sup):
                b = k_ % nbuf
                if k_ + depth < sup:
                    for d_ in sd[(k_ + depth) % nbuf]:
                        d_.wait()
                    sd[(k_ + depth) % nbuf] = []
                    gd[k_ + depth] = gfire(k_ + depth)
                for d_ in gd[k_]:
                    d_.wait()
                sd[b] = sfire(k_)
            for b in range(nbuf):
                for d_ in sd[b]:
                    d_.wait()
            return 0
        lax.fori_loop(0, js, body, 0)
        plsc.subcore_barrier()
        for i in range(nper // CH):
            r0 = s * nper + i * CH
            pltpu.sync_copy(acc.at[pl.ds(r0, CH)], p_hbm.at[c, pl.ds(r0, CH)])
        if nper % CH:
            r0 = s * nper + (nper // CH) * CH
            pltpu.sync_copy(acc.at[pl.ds(r0, nper % CH)],
                            p_hbm.at[c, pl.ds(r0, nper % CH)])
        if with_scalar:
            pltpu.sync_copy(sacc.at[pl.ds(s * nper, nper)],
                            sa_hbm.at[c, pl.ds(s * nper, nper)])

    kfn = functools.partial(pl.kernel, mesh=mesh, out_type=out_type,
                            scratch_types=scratch)(k)
    if with_scalar:
        return kfn(yp, er, dinvp)
    return kfn(yp, er)


def _tc_scale(dinvp, xp, npad, d, bn=1024):
    g = npad // bn

    def body(dv, xr, o):
        o[...] = dv[...][:, None] * xr[...]

    return pl.pallas_call(
        body, grid=(g,),
        in_specs=[pl.BlockSpec((bn,), lambda i: (i,)),
                  pl.BlockSpec((bn, d), lambda i: (i, 0))],
        out_specs=pl.BlockSpec((bn, d), lambda i: (i, 0)),
        out_shape=jax.ShapeDtypeStruct((npad, d), jnp.float32),
    )(dinvp, xp)


def _tc_mlp(p0, p1, xp, dinvp, dinv2p, sp, W1, b1, W2, b2, npad, bn=1024):
    g = npad // bn
    d_in = xp.shape[1]
    d_h, d_out = W1.shape[0], W2.shape[0]

    def body(p0r, p1r, xr, dvr, d2r, sr, w1r, b1r, w2r, b2r, g_ref, y1_ref):
        dv = dvr[...][:, None]
        d2 = d2r[...][:, None]
        t = dv * (p0r[...] + p1r[...]) + d2 * xr[...]
        h = lax.dot_general(t, w1r[...], (((1,), (1,)), ((), ())),
                            preferred_element_type=jnp.float32)
        h = jnp.maximum(h + sr[...][:, None] * b1r[...][None, :], 0.0)
        gm = lax.dot_general(h, w2r[...], (((1,), (1,)), ((), ())),
                             preferred_element_type=jnp.float32)
        gm = gm + b2r[...][None, :]
        g_ref[...] = gm
        y1_ref[...] = dv * gm

    return pl.pallas_call(
        body, grid=(g,),
        in_specs=[pl.BlockSpec((bn, d_in), lambda i: (i, 0)),
                  pl.BlockSpec((bn, d_in), lambda i: (i, 0)),
                  pl.BlockSpec((bn, d_in), lambda i: (i, 0)),
                  pl.BlockSpec((bn,), lambda i: (i,)),
                  pl.BlockSpec((bn,), lambda i: (i,)),
                  pl.BlockSpec((bn,), lambda i: (i,)),
                  pl.BlockSpec((d_h, d_in), lambda i: (0, 0)),
                  pl.BlockSpec((d_h,), lambda i: (0,)),
                  pl.BlockSpec((d_out, d_h), lambda i: (0, 0)),
                  pl.BlockSpec((d_out,), lambda i: (0,))],
        out_specs=[pl.BlockSpec((bn, d_out), lambda i: (i, 0)),
                   pl.BlockSpec((bn, d_out), lambda i: (i, 0))],
        out_shape=[jax.ShapeDtypeStruct((npad, d_out), jnp.float32),
                   jax.ShapeDtypeStruct((npad, d_out), jnp.float32)],
    )(p0, p1, xp, dinvp, dinv2p, sp, W1, b1, W2, b2)


def _tc_combine(p0, p1, gm, dinvp, dinv2p, npad, d, bn=1024):
    g = npad // bn

    def body(p0r, p1r, gr, dvr, d2r, o):
        o[...] = (dvr[...][:, None] * (p0r[...] + p1r[...])
                  + d2r[...][:, None] * gr[...])

    return pl.pallas_call(
        body, grid=(g,),
        in_specs=[pl.BlockSpec((bn, d), lambda i: (i, 0)),
                  pl.BlockSpec((bn, d), lambda i: (i, 0)),
                  pl.BlockSpec((bn, d), lambda i: (i, 0)),
                  pl.BlockSpec((bn,), lambda i: (i,)),
                  pl.BlockSpec((bn,), lambda i: (i,))],
        out_specs=pl.BlockSpec((bn, d), lambda i: (i, 0)),
        out_shape=jax.ShapeDtypeStruct((npad, d), jnp.float32),
    )(p0, p1, gm, dinvp, dinv2p)


def kernel(x, edge_index, W1, b1, W2, b2):
    n, d_in = x.shape
    e = edge_index.shape[1]
    nw = NC * NS
    sup = SUP
    assert e % (nw * CH * sup) == 0, "edge count must tile evenly"
    ept = e // nw
    js = ept // (CH * sup)
    npad = -(-(n + 1) // 2048) * 2048
    nper = npad // NS

    er = edge_index.reshape(2, NC, NS, js, sup, CH)

    xp = jnp.pad(x, ((0, npad - n), (0, 0)))

    cnt = _sc_degree(er, npad, nper)
    deg = cnt[0, :n] + cnt[1, :n] + 1.0
    dinv = lax.rsqrt(deg)
    dinv2 = 1.0 / deg
    dinvp = jnp.pad(dinv, (0, npad - n))
    dinv2p = jnp.pad(dinv2, (0, npad - n))

    y0 = _tc_scale(dinvp, xp, npad, d_in)
    p, sa = _sc_propagate(y0, er, dinvp, npad, nper, True)
    s = dinv * (sa[0, :n] + sa[1, :n]) + dinv2
    sp = jnp.pad(s, (0, npad - n))

    g, y1 = _tc_mlp(p[0], p[1], xp, dinvp, dinv2p, sp, W1, b1, W2, b2, npad)

    p2 = _sc_propagate(y1, er, None, npad, nper, False)[0]
    out = _tc_combine(p2[0], p2[1], g, dinvp, dinv2p, npad, x.shape[1])
    return out[:n]
